# Initial kernel scaffold; baseline (speedup 1.0000x reference)
import jax
import jax.numpy as jnp
from jax.experimental import pallas as pl


def kernel(X, x_ref):
    raise NotImplementedError("write your pallas kernel here")


# speedup vs baseline: 2.3147x; 2.3147x over previous
import functools
import math

import jax
import jax.numpy as jnp
from jax.experimental import pallas as pl
from jax.experimental.pallas import tpu as pltpu

_BIG = 1e30
_KNN = 10


def _body(x_ref_blk, xr_blk, out_ref, run_ref, *, nblocks, n_valid, qb, nb):
    c = pl.program_id(1)

    @pl.when(c == 0)
    def _init():
        run_ref[...] = jnp.full((qb, 16), _BIG, jnp.float32)

    xb = x_ref_blk[...]
    xrb = xr_blk[...]
    dot = jax.lax.dot_general(
        xb * (-2.0), xrb, (((1,), (1,)), ((), ())),
        preferred_element_type=jnp.float32)
    r2 = jnp.sum(xrb * xrb, axis=1)
    nglob = c * nb + jax.lax.broadcasted_iota(jnp.int32, (1, nb), 1)
    r2m = jnp.where(nglob < n_valid, r2[None, :], _BIG)
    s = dot + r2m

    w = jnp.concatenate([s, run_ref[...]], axis=1)
    lane = jax.lax.broadcasted_iota(jnp.int32, (qb, nb + 16), 1)
    vals = []
    for _ in range(_KNN):
        m = jnp.min(w, axis=1, keepdims=True)
        eq = w == m
        idx = jnp.min(jnp.where(eq, lane, jnp.int32(2 ** 30)), axis=1,
                      keepdims=True)
        w = jnp.where(lane == idx, _BIG, w)
        vals.append(m)
    newrun = jnp.concatenate(vals + [jnp.full((qb, 16 - _KNN), _BIG)], axis=1)
    run_ref[...] = newrun

    @pl.when(c == nblocks - 1)
    def _finish():
        x2 = jnp.sum(xb * xb, axis=1)
        d2 = newrun[:, _KNN - 1] + x2
        out_ref[0, 0, :] = jnp.sqrt(jnp.maximum(d2, 0.0))


def kernel(X, x_ref):
    q, d = X.shape
    n = x_ref.shape[0]
    qb = 512 if q % 512 == 0 else q
    nb = 1024
    nblocks = math.ceil(n / nb)
    npad = nblocks * nb
    xr = jnp.pad(x_ref, ((0, npad - n), (0, 0)))
    r = q // qb

    out = pl.pallas_call(
        functools.partial(_body, nblocks=nblocks, n_valid=n, qb=qb, nb=nb),
        grid=(r, nblocks),
        in_specs=[
            pl.BlockSpec((qb, d), lambda i, j: (i, 0)),
            pl.BlockSpec((nb, d), lambda i, j: (j, 0)),
        ],
        out_specs=pl.BlockSpec((1, 1, qb), lambda i, j: (i, 0, 0)),
        out_shape=jax.ShapeDtypeStruct((r, 1, qb), jnp.float32),
        scratch_shapes=[pltpu.VMEM((qb, 16), jnp.float32)],
    )(X, xr)
    return out.reshape(q)


# speedup vs baseline: 15.5133x; 6.7021x over previous
import functools
import math

import jax
import jax.numpy as jnp
from jax import lax
from jax.experimental import pallas as pl
from jax.experimental.pallas import tpu as pltpu
from jax.experimental.pallas import tpu_sc as plsc

_KNN = 10


def _oems_pairs(n):
    res = []
    def merge(lo, hi, r):
        step = r * 2
        if step < hi - lo:
            merge(lo, hi, step)
            merge(lo + r, hi, step)
            for i in range(lo + r, hi - r, step):
                res.append((i, i + r))
        else:
            res.append((lo, lo + r))
    def sort(lo, hi):
        if hi - lo >= 1:
            mid = lo + (hi - lo) // 2
            sort(lo, mid)
            sort(mid + 1, hi)
            merge(lo, hi, 1)
    sort(0, n - 1)
    return res


def _bitonic_merge_pairs(n):
    res = []
    d = n // 2
    while d >= 1:
        for i in range(n):
            if i % (2 * d) < d:
                res.append((i, i + d))
        d //= 2
    return res


_P16 = _oems_pairs(16)
_BM16 = _bitonic_merge_pairs(16)


def _ce(ws, pairs):
    for a, b in pairs:
        lo = jnp.minimum(ws[a], ws[b])
        hi = jnp.maximum(ws[a], ws[b])
        ws[a], ws[b] = lo, hi


def _body_a(x_blk, xr_blk, out_ref, *, nblocks, qb, nb):
    c = pl.program_id(1)
    xb = x_blk[...]
    xrb = xr_blk[...]
    dot = jax.lax.dot_general(
        xrb, xb * jnp.bfloat16(-2.0), (((1,), (1,)), ((), ())),
        preferred_element_type=jnp.float32)
    xrf = xrb.astype(jnp.float32)
    r2 = jnp.sum(xrf * xrf, axis=1)
    s = dot.astype(jnp.bfloat16) + r2[:, None].astype(jnp.bfloat16)

    m0 = nb // 16
    ws = [s[i * m0:(i + 1) * m0, :] for i in range(16)]
    _ce(ws, _P16)
    h = m0
    while h > 1:
        hh = h // 2
        lo = [w[:hh, :] for w in ws]
        hi = [w[hh:, :] for w in ws]
        ws = [jnp.minimum(lo[i], hi[15 - i]) for i in range(16)]
        _ce(ws, _BM16)
        h = hh
    xf = xb.astype(jnp.float32)
    x2 = jnp.sum(xf * xf, axis=1)[None, :]
    lists = jnp.concatenate(ws, axis=0).astype(jnp.float32)
    dists = jnp.sqrt(jnp.maximum(lists + x2, 0.0))
    out_ref[0, pl.ds(c * 16, 16), :] = dists


def _stage_a(X, x_ref, qb, nb):
    q, d = X.shape
    n = x_ref.shape[0]
    nblocks = math.ceil(n / nb)
    npad = nblocks * nb
    pad = jnp.zeros((npad - n, d), x_ref.dtype).at[:, 0].set(1e19)
    xr = jnp.concatenate([x_ref, pad], axis=0).astype(jnp.bfloat16)
    xc = X.astype(jnp.bfloat16)
    r = q // qb
    c16 = nblocks * 16
    out = pl.pallas_call(
        functools.partial(_body_a, nblocks=nblocks, qb=qb, nb=nb),
        grid=(r, nblocks),
        in_specs=[
            pl.BlockSpec((qb, d), lambda i, j: (i, 0)),
            pl.BlockSpec((nb, d), lambda i, j: (j, 0)),
        ],
        out_specs=pl.BlockSpec((1, c16, qb), lambda i, j: (i, 0, 0)),
        out_shape=jax.ShapeDtypeStruct((r, c16, qb), jnp.float32),
        compiler_params=pltpu.CompilerParams(
            dimension_semantics=("parallel", "arbitrary")),
    )(xc, xr)
    return out


def _stage_b(cand):
    r, c16, qb = cand.shape
    q = r * qb
    nlists = c16 // 16
    nc = 2
    nw = 32
    qpw = q // nw
    sub_per_rb = qb // qpw
    ngroups = qpw // 16
    mesh = plsc.VectorSubcoreMesh(core_axis_name="c", subcore_axis_name="s")

    @functools.partial(
        pl.kernel, mesh=mesh,
        out_type=jax.ShapeDtypeStruct((q,), jnp.float32),
        scratch_types=[
            pltpu.VMEM((c16, qpw), jnp.float32),
            pltpu.VMEM((qpw,), jnp.float32),
        ],
    )
    def k(cand_hbm, out_hbm, cv, res_v):
        wid = lax.axis_index("s") * nc + lax.axis_index("c")
        rb = wid // sub_per_rb
        qoff = (wid % sub_per_rb) * qpw
        pltpu.sync_copy(cand_hbm.at[rb, :, pl.ds(qoff, qpw)], cv)

        def per_group(g, carry):
            col = g * 16
            run = [cv[i, pl.ds(col, 16)] for i in range(16)]
            for c in range(1, nlists):
                blk = [cv[c * 16 + i, pl.ds(col, 16)] for i in range(16)]
                run = [jnp.minimum(run[i], blk[15 - i]) for i in range(16)]
                _ce(run, _BM16)
            res_v[pl.ds(col, 16)] = run[_KNN - 1]
            return carry

        lax.fori_loop(0, ngroups, per_group, 0)
        pltpu.sync_copy(res_v, out_hbm.at[pl.ds(wid * qpw, qpw)])

    return k(cand)


def kernel(X, x_ref):
    cand = _stage_a(X, x_ref, qb=1024, nb=8192)
    return _stage_b(cand)
